# baseline (speedup 1.0000x reference)
import jax
import jax.numpy as jnp
from jax.experimental import pallas as pl
from jax.experimental.pallas import tpu as pltpu

N = 10000
E = 160000
H = 8
F = 128
DEPTH = 6


def _leaky(x, s=0.01):
    return jnp.where(x >= 0, x, s * x)


def _gat(x, src, dst, e, Wn, We, a_s, a_d):
    n = x.shape[0]
    h = (x @ Wn.T).reshape(n, H, F)
    et = e.reshape(-1, 1) @ We.T
    sh = h[src]
    dh = h[dst]
    attn = (sh * a_s).sum(-1) + (dh * a_d).sum(-1) + et
    attn = _leaky(attn)
    m = jax.ops.segment_max(attn, dst, num_segments=n)
    m = jnp.where(jnp.isfinite(m), m, 0.0)
    ex = jnp.exp(attn - m[dst])
    den = jax.ops.segment_sum(ex, dst, num_segments=n)
    alpha = ex / den[dst]
    hn = jax.ops.segment_sum(sh * alpha[..., None], dst, num_segments=n)
    return _leaky(hn.mean(axis=1))


def _dot_hi(a, b):
    return jax.lax.dot_general(a, b, (((1,), (0,)), ((), ())),
                               precision=jax.lax.Precision.HIGHEST,
                               preferred_element_type=jnp.float32)


def _mlp_body(hg_ref, w1_ref, b1_ref, w2_ref, b2_ref, out_ref):
    hg = hg_ref[...]
    hidden = _leaky(_dot_hi(hg, w1_ref[...]) + b1_ref[...])
    out_ref[...] = _dot_hi(hidden, w2_ref[...]) + b2_ref[...]


def _mlp_pallas(hg, W1, b1, W2, b2):
    return pl.pallas_call(
        _mlp_body,
        out_shape=jax.ShapeDtypeStruct((1, 1), jnp.float32),
    )(hg, W1.T, b1.reshape(1, -1), W2.T, b2.reshape(1, 1))


def kernel(x1, edge_index1, e1, x2, edge_index2, e2, x3, edge_index3, e3,
           x4, edge_index4, e4, x5, edge_index5, e5, x6, edge_index6, e6,
           ratio, temp,
           Wn0, We0, as0, ad0, Wn1, We1, as1, ad1, Wn2, We2, as2, ad2,
           Wn3, We3, as3, ad3, Wn4, We4, as4, ad4, Wn5, We5, as5, ad5,
           Wg1, bg1, Wr, br, Wt, bt, W1, b1, W2, b2):
    kw = dict(locals())
    hs = [kw[f"x{g}"] for g in range(1, 7)]
    ints = [kw[f"edge_index{g}"] for g in range(1, 7)]
    for l in range(DEPTH):
        Wn, We = kw[f"Wn{l}"], kw[f"We{l}"]
        a_s, a_d = kw[f"as{l}"], kw[f"ad{l}"]
        hs = [
            _gat(hs[gi], ints[gi][0], ints[gi][1], kw[f"e{gi + 1}"], Wn, We, a_s, a_d)
            for gi in range(6)
        ]
    hgs = [hs[gi].sum(axis=0, keepdims=True) @ Wg1.T + bg1 for gi in range(6)]
    hr = _leaky(ratio @ Wr.T + br)
    ht = _leaky(temp @ Wt.T + bt)
    hg = jnp.concatenate(hgs + [hr, ht], axis=1)
    return _mlp_pallas(hg, W1, b1, W2, b2)


# speedup vs baseline: 11.2082x; 11.2082x over previous
import functools

import jax
import jax.numpy as jnp
from jax import lax
from jax.experimental import pallas as pl
from jax.experimental.pallas import tpu as pltpu
from jax.experimental.pallas import tpu_sc as plsc

N = 10000
NP = 10240
E = 160000
H = 8
F = 128
DEPTH = 6
G = 6
NCORES = 2
NTILES = 16
NG2 = G // NCORES
EPT = 10240
NFAKE = N + 64
CH = 64
NCH = EPT // CH
CHH = 8
GRP = 8
ROWS_PT = NP // NTILES
XCH = 8

_f32 = jnp.float32


def _leaky_v(x):
    return jnp.where(x >= 0, x, 0.01 * x)


def _dot_hi(a, b):
    return jax.lax.dot_general(a, b, (((1,), (0,)), ((), ())),
                               precision=jax.lax.Precision.HIGHEST,
                               preferred_element_type=_f32)


def _mm_body(x_ref, wh_ref, wsd_ref, h_ref, sd_ref):
    x = x_ref[...]
    h_ref[...] = _dot_hi(x, wh_ref[...])
    sd_ref[...] = _dot_hi(x, wsd_ref[...])


def _project(x, wh, wsd):
    m, dp = x.shape
    bm = 1000
    return pl.pallas_call(
        _mm_body,
        grid=(m // bm,),
        in_specs=[pl.BlockSpec((bm, dp), lambda i: (i, 0)),
                  pl.BlockSpec((dp, 1024), lambda i: (0, 0)),
                  pl.BlockSpec((dp, F), lambda i: (0, 0))],
        out_specs=[pl.BlockSpec((bm, 1024), lambda i: (i, 0)),
                   pl.BlockSpec((bm, F), lambda i: (i, 0))],
        out_shape=[jax.ShapeDtypeStruct((m, 1024), _f32),
                   jax.ShapeDtypeStruct((m, F), _f32)],
    )(x, wh, wsd)


def _bounds_body(sd_ref, e_ref, wa_ref, c_ref):
    m_all = jnp.max(sd_ref[...], axis=0, keepdims=True)
    ms = m_all[:, 0:16]
    md = m_all[:, 16:32]
    me = jnp.max(jnp.abs(e_ref[...]))
    c0 = ms + md + me * wa_ref[...]
    c_ref[...] = _leaky_v(c0)[None]


def _bounds(sd, ev, weabs):
    out = pl.pallas_call(
        _bounds_body,
        grid=(G,),
        in_specs=[pl.BlockSpec((N, F), lambda i: (i, 0)),
                  pl.BlockSpec((1, 1, E), lambda i: (i, 0, 0)),
                  pl.BlockSpec((1, 16), lambda i: (0, 0))],
        out_specs=pl.BlockSpec((1, 1, 16), lambda i: (i, 0, 0)),
        out_shape=jax.ShapeDtypeStruct((G, 1, 16), _f32),
    )(sd, ev.reshape(G, 1, E), weabs)
    return out.reshape(G, 16)


def _edge_body(ht, sdt, srcg, dstg, dstl, srch, evb, cb,
               xn, exb, denb,
               tab_sh,
               srcv8, dgv8, dlv8, srchv,
               evv1, ex16b,
               srows, drows, denrows,
               hrows, msg, xbuf, cbuf, sem):
    cix = lax.axis_index("c")
    six = lax.axis_index("s")
    denb_c = denb.at[cix]

    def _zero_xbuf():
        def zx(i, c):
            for k in range(8):
                xbuf[i, pl.ds(k * 16, 16)] = jnp.zeros((16,), _f32)
            return c
        lax.fori_loop(0, XCH, zx, 0)

    def _zero_table():
        _zero_xbuf()

        def zt(k, c):
            base = pl.multiple_of(six * ROWS_PT + k * XCH, 8)
            pltpu.sync_copy(xbuf, tab_sh.at[pl.ds(base, XCH)])
            return c
        lax.fori_loop(0, ROWS_PT // XCH, zt, 0)

    def round_body(r, carry):
        g = cix * NG2 + r
        pltpu.sync_copy(cb.at[g, 0], cbuf)
        _zero_table()

        def zm(i, c):
            for k in range(8):
                msg[i, pl.ds(k * 16, 16)] = jnp.zeros((16,), _f32)
            return c
        lax.fori_loop(0, CH, zm, 0)
        plsc.subcore_barrier()

        cv = cbuf[pl.ds(0, 16)]
        wev = cbuf[pl.ds(16, 16)]
        evbase = (g * NTILES + six) * (EPT * 16)
        exbase = ((cix * NG2 + r) * NTILES + six) * (EPT * 16)

        def p1(jg, c):
            pltpu.sync_copy(srcg.at[g, six, pl.ds(jg * GRP, GRP)], srcv8)
            pltpu.sync_copy(dstg.at[g, six, pl.ds(jg * GRP, GRP)], dgv8)
            pltpu.sync_copy(dstl.at[g, six, pl.ds(jg * GRP, GRP)], dlv8)

            def p1j(jj, cj):
                j = jg * GRP + jj
                pltpu.async_copy(sdt.at[srcv8.at[jj, 0]], srows, sem).wait()
                pltpu.async_copy(sdt.at[dgv8.at[jj, 0]], drows, sem).wait()
                off = pl.multiple_of(evbase + j * (CH * 16), 8)
                pltpu.sync_copy(evb.at[pl.ds(off, CH * 16)], evv1)

                def pe(e2, c2):
                    a = (srows[e2, pl.ds(0, 16)] + drows[e2, pl.ds(16, 16)]
                         + evv1[pl.ds(e2 * 16, 16)] * wev)
                    a = jnp.where(a >= 0, a, 0.01 * a)
                    e_val = jnp.exp(a - cv)
                    msg[e2, pl.ds(0, 16)] = e_val
                    ex16b[pl.ds(e2 * 16, 16)] = e_val
                    return c2
                lax.fori_loop(0, CH, pe, 0)
                off2 = pl.multiple_of(exbase + j * (CH * 16), 8)
                pltpu.sync_copy(ex16b, exb.at[pl.ds(off2, CH * 16)])
                pltpu.sync_copy(msg, tab_sh.at[dlv8.at[jj, 0]], add=True)
                return cj
            lax.fori_loop(0, GRP, p1j, 0)
            return c
        lax.fori_loop(0, NCH // GRP, p1, 0)
        plsc.subcore_barrier()

        def dcp(k, c):
            base = pl.multiple_of(six * ROWS_PT + k * XCH, 8)
            pltpu.sync_copy(tab_sh.at[pl.ds(base, XCH)], xbuf)
            pltpu.sync_copy(xbuf, denb_c.at[pl.ds(base, XCH)])
            return c
        lax.fori_loop(0, ROWS_PT // XCH, dcp, 0)
        plsc.subcore_barrier()
        _zero_table()
        plsc.subcore_barrier()

        def p2(jg, c):
            pltpu.sync_copy(dstl.at[g, six, pl.ds(jg * GRP, GRP)], dlv8)

            def p2j(jj, cj):
                j = jg * GRP + jj
                pltpu.async_copy(denb_c.at[dlv8.at[jj, 0]], denrows, sem).wait()
                off = pl.multiple_of(exbase + j * (CH * 16), 8)
                pltpu.sync_copy(exb.at[pl.ds(off, CH * 16)], ex16b)
                pltpu.sync_copy(srch.at[g, six, pl.ds(j * (CH // CHH), CH // CHH)],
                                srchv)

                def p2k(kk, ck):
                    pltpu.async_copy(ht.at[srchv.at[kk, 0]], hrows, sem).wait()

                    def pm(e3, c2):
                        e2 = kk * CHH + e3
                        av = (ex16b[pl.ds(e2 * 16, 16)]
                              / denrows[e2, pl.ds(0, 16)] * 0.125)
                        accs = [jnp.zeros((16,), _f32) for _ in range(8)]
                        for h in range(H):
                            ah = av[h]
                            for k in range(8):
                                accs[k] = accs[k] + ah * hrows[e3, pl.ds(h * F + k * 16, 16)]
                        for k in range(8):
                            msg[e2, pl.ds(k * 16, 16)] = accs[k]
                        return c2
                    lax.fori_loop(0, CHH, pm, 0)
                    return ck
                lax.fori_loop(0, CH // CHH, p2k, 0)
                pltpu.sync_copy(msg, tab_sh.at[dlv8.at[jj, 0]], add=True)
                return cj
            lax.fori_loop(0, GRP, p2j, 0)
            return c
        lax.fori_loop(0, NCH // GRP, p2, 0)
        plsc.subcore_barrier()

        def cpo(k, c):
            base = pl.multiple_of(six * ROWS_PT + k * XCH, 8)

            @pl.when(base < N)
            def _copy_out():
                pltpu.sync_copy(tab_sh.at[pl.ds(base, XCH)], xbuf)

                def lk(i, c2):
                    for kk in range(8):
                        v = xbuf[i, pl.ds(kk * 16, 16)]
                        xbuf[i, pl.ds(kk * 16, 16)] = jnp.where(v >= 0, v, 0.01 * v)
                    return c2
                lax.fori_loop(0, XCH, lk, 0)
                pltpu.sync_copy(xbuf, xn.at[pl.ds(pl.multiple_of(g * N + base, 8),
                                                  XCH)])
            return c
        lax.fori_loop(0, ROWS_PT // XCH, cpo, 0)
        plsc.subcore_barrier()
        return carry

    lax.fori_loop(0, NG2, round_body, 0)


def _edge_call(ht, sdt, srcg, dstg, dstl, srch, evb, cb):
    mesh = plsc.VectorSubcoreMesh(core_axis_name="c", subcore_axis_name="s",
                                  num_cores=NCORES, num_subcores=NTILES)
    fn = pl.kernel(
        _edge_body,
        out_type=[jax.ShapeDtypeStruct((G * N, F), _f32),
                  jax.ShapeDtypeStruct((NCORES * NG2 * NTILES * EPT * 16,), _f32),
                  jax.ShapeDtypeStruct((NCORES, NP, F), _f32)],
        mesh=mesh,
        scratch_types=[
            pltpu.VMEM_SHARED((NP, F), _f32),
            pltpu.VMEM((GRP, 1, CH), jnp.int32),
            pltpu.VMEM((GRP, 1, CH), jnp.int32),
            pltpu.VMEM((GRP, 1, CH), jnp.int32),
            pltpu.VMEM((CH // CHH, 1, CHH), jnp.int32),
            pltpu.VMEM((CH * 16,), _f32),
            pltpu.VMEM((CH * 16,), _f32),
            pltpu.VMEM((CH, F), _f32),
            pltpu.VMEM((CH, F), _f32),
            pltpu.VMEM((CH, F), _f32),
            pltpu.VMEM((CHH, 1024), _f32),
            pltpu.VMEM((CH, F), _f32),
            pltpu.VMEM((XCH, F), _f32),
            pltpu.VMEM((32,), _f32),
            pltpu.SemaphoreType.DMA,
        ],
    )
    return fn(ht, sdt, srcg, dstg, dstl, srch, evb, cb)


def _rsum_body(x_ref, o_ref):
    o_ref[...] = jnp.sum(x_ref[...], axis=0, keepdims=True)[None]


def _rsum(x):
    out = pl.pallas_call(
        _rsum_body,
        grid=(G,),
        in_specs=[pl.BlockSpec((N, F), lambda i: (i, 0))],
        out_specs=pl.BlockSpec((1, 1, F), lambda i: (i, 0, 0)),
        out_shape=jax.ShapeDtypeStruct((G, 1, F), _f32),
    )(x)
    return out.reshape(G, F)


def _head_body(sums_ref, wg1t_ref, bg1_ref, ratio_ref, wrt_ref, br_ref,
               temp_ref, wtt_ref, bt_ref, w1g_ref, w1r_ref, w1t_ref, b1_ref,
               w2t_ref, b2_ref, out_ref):
    hgs = _dot_hi(sums_ref[...], wg1t_ref[...]) + bg1_ref[...]
    hp = _dot_hi(hgs[0:1, :], w1g_ref[0])
    for g in range(1, G):
        hp = hp + _dot_hi(hgs[g:g + 1, :], w1g_ref[g])
    hr = _leaky_v(_dot_hi(ratio_ref[...], wrt_ref[...]) + br_ref[...])
    ht = _leaky_v(_dot_hi(temp_ref[...], wtt_ref[...]) + bt_ref[...])
    hp = hp + _dot_hi(hr, w1r_ref[...]) + _dot_hi(ht, w1t_ref[...])
    hidden = _leaky_v(hp + b1_ref[...])
    out_ref[...] = _dot_hi(hidden, w2t_ref[...]) + b2_ref[...]


def _head(sums, Wg1, bg1, ratio, Wr, br, temp, Wt, bt, W1, b1, W2, b2):
    w1g = W1[:, :G * 64].T.reshape(G, 64, 512)
    w1r = W1[:, G * 64:G * 64 + 6].T
    w1t = W1[:, G * 64 + 6:].T
    return pl.pallas_call(
        _head_body,
        out_shape=jax.ShapeDtypeStruct((1, 1), _f32),
    )(sums, Wg1.T, bg1.reshape(1, -1), ratio, Wr.T, br.reshape(1, -1),
      temp, Wt.T, bt.reshape(1, -1), w1g, w1r, w1t, b1.reshape(1, -1),
      W2.T, b2.reshape(1, 1))


def kernel(x1, edge_index1, e1, x2, edge_index2, e2, x3, edge_index3, e3,
           x4, edge_index4, e4, x5, edge_index5, e5, x6, edge_index6, e6,
           ratio, temp,
           Wn0, We0, as0, ad0, Wn1, We1, as1, ad1, Wn2, We2, as2, ad2,
           Wn3, We3, as3, ad3, Wn4, We4, as4, ad4, Wn5, We5, as5, ad5,
           Wg1, bg1, Wr, br, Wt, bt, W1, b1, W2, b2):
    kw = dict(locals())
    xs = [kw[f"x{g}"] for g in range(1, 7)]
    eis = [kw[f"edge_index{g}"] for g in range(1, 7)]
    evs = [kw[f"e{g}"] for g in range(1, 7)]

    pad_pt = EPT - E // NTILES

    def _ept_pad(a, val):
        a = a.reshape(G, NTILES, E // NTILES)
        return jnp.pad(a, ((0, 0), (0, 0), (0, pad_pt)), constant_values=val)

    src_p = _ept_pad(jnp.stack([eis[g][0] + g * N for g in range(G)]), 0)
    srcg = src_p.reshape(G, NTILES, NCH, 1, CH)
    srch = src_p.reshape(G, NTILES, EPT // CHH, 1, CHH)
    dstg = _ept_pad(jnp.stack([eis[g][1] + g * N for g in range(G)]), 0)
    dstg = dstg.reshape(G, NTILES, NCH, 1, CH)
    dstl = _ept_pad(jnp.stack([eis[g][1] for g in range(G)]), NFAKE)
    dstl = dstl.reshape(G, NTILES, NCH, 1, CH)
    ev_flat = jnp.stack(evs)
    ev_p = _ept_pad(ev_flat, 0.0)
    evb = jnp.broadcast_to(ev_p[..., None],
                           (G, NTILES, EPT, 16)).reshape(-1)

    x = jnp.concatenate(xs, axis=0)
    x = jnp.pad(x, ((0, 0), (0, 15)))

    for l in range(DEPTH):
        Wn, We = kw[f"Wn{l}"], kw[f"We{l}"]
        a_s, a_d = kw[f"as{l}"], kw[f"ad{l}"]
        dp = x.shape[1]
        d0 = Wn.shape[1]
        wn3 = Wn.reshape(H, F, d0)
        ws = jnp.einsum("hf,hfd->hd", a_s, wn3)
        wd = jnp.einsum("hf,hfd->hd", a_d, wn3)
        pad = ((0, dp - d0), (0, 0))
        wh = jnp.pad(Wn.T, pad)
        ws2 = jnp.concatenate([ws, ws], 0).T
        wd2 = jnp.concatenate([wd, wd], 0).T
        wsd = jnp.pad(jnp.concatenate([ws2, wd2], 1),
                      ((0, dp - d0), (0, F - 32)))
        we16 = jnp.tile(We[:, 0], 2)
        weabs = jnp.abs(we16).reshape(1, 16)

        ht, sd = _project(x, wh, wsd)
        cb8 = _bounds(sd, ev_flat, weabs)
        cb = jnp.concatenate(
            [cb8, jnp.broadcast_to(we16, (G, 16))], axis=1).reshape(G, 1, 32)
        x, _, _ = _edge_call(ht, sd, srcg, dstg, dstl, srch, evb, cb)

    sums = _rsum(x)
    return _head(sums, Wg1, bg1, ratio, Wr, br, temp, Wt, bt, W1, b1, W2, b2)


# speedup vs baseline: 13.7454x; 1.2264x over previous
import functools

import jax
import jax.numpy as jnp
from jax import lax
from jax.experimental import pallas as pl
from jax.experimental.pallas import tpu as pltpu
from jax.experimental.pallas import tpu_sc as plsc

N = 10000
NP = 10240
E = 160000
H = 8
F = 128
DEPTH = 6
G = 6
NCORES = 2
NTILES = 16
NG2 = G // NCORES
EPT = 10240
NFAKE = N + 64
CH = 64
NCH = EPT // CH
CHH = 8
GRP = 8
ROWS_PT = NP // NTILES
XCH = 8

_f32 = jnp.float32


def _leaky_v(x):
    return jnp.where(x >= 0, x, 0.01 * x)


def _dot_hi(a, b):
    return jax.lax.dot_general(a, b, (((1,), (0,)), ((), ())),
                               precision=jax.lax.Precision.HIGHEST,
                               preferred_element_type=_f32)


def _mm_body(x_ref, wh_ref, wsd_ref, h_ref, sd_ref):
    x = x_ref[...]
    h_ref[...] = _dot_hi(x, wh_ref[...])
    sd_ref[...] = _dot_hi(x, wsd_ref[...])


def _project(x, wh, wsd):
    m, dp = x.shape
    bm = 1000
    return pl.pallas_call(
        _mm_body,
        grid=(m // bm,),
        in_specs=[pl.BlockSpec((bm, dp), lambda i: (i, 0)),
                  pl.BlockSpec((dp, 1024), lambda i: (0, 0)),
                  pl.BlockSpec((dp, F), lambda i: (0, 0))],
        out_specs=[pl.BlockSpec((bm, 1024), lambda i: (i, 0)),
                   pl.BlockSpec((bm, F), lambda i: (i, 0))],
        out_shape=[jax.ShapeDtypeStruct((m, 1024), _f32),
                   jax.ShapeDtypeStruct((m, F), _f32)],
    )(x, wh, wsd)


def _bounds_body(sd_ref, e_ref, wa_ref, c_ref):
    m_all = jnp.max(sd_ref[...], axis=0, keepdims=True)
    ms = m_all[:, 0:16]
    md = m_all[:, 16:32]
    me = jnp.max(jnp.abs(e_ref[...]))
    c0 = ms + md + me * wa_ref[...]
    c_ref[...] = _leaky_v(c0)[None]


def _bounds(sd, ev, weabs):
    out = pl.pallas_call(
        _bounds_body,
        grid=(G,),
        in_specs=[pl.BlockSpec((N, F), lambda i: (i, 0)),
                  pl.BlockSpec((1, 1, E), lambda i: (i, 0, 0)),
                  pl.BlockSpec((1, 16), lambda i: (0, 0))],
        out_specs=pl.BlockSpec((1, 1, 16), lambda i: (i, 0, 0)),
        out_shape=jax.ShapeDtypeStruct((G, 1, 16), _f32),
    )(sd, ev.reshape(G, 1, E), weabs)
    return out.reshape(G, 16)


def _edge_body(ht, sdt, srcg, dstg, dstl, srch, evb, cb,
               xn, exb, denb,
               tab_sh,
               srcv8, dgv8, dlv8, srchv,
               evv1, ex16b,
               srows, denrows,
               hrows, hrows2, msg, xbuf, cbuf, sem, sem2):
    cix = lax.axis_index("c")
    six = lax.axis_index("s")
    denb_c = denb.at[cix]

    def _zero_xbuf():
        def zx(i, c):
            for k in range(8):
                xbuf[i, pl.ds(k * 16, 16)] = jnp.zeros((16,), _f32)
            return c
        lax.fori_loop(0, XCH, zx, 0)

    def _zero_table():
        _zero_xbuf()

        def zt(k, c):
            base = pl.multiple_of(six * ROWS_PT + k * XCH, 8)
            pltpu.sync_copy(xbuf, tab_sh.at[pl.ds(base, XCH)])
            return c
        lax.fori_loop(0, ROWS_PT // XCH, zt, 0)

    def round_body(r, carry):
        g = cix * NG2 + r
        pltpu.sync_copy(cb.at[g, 0], cbuf)
        _zero_table()

        def zm(i, c):
            for k in range(8):
                msg[i, pl.ds(k * 16, 16)] = jnp.zeros((16,), _f32)
            return c
        lax.fori_loop(0, CH, zm, 0)
        plsc.subcore_barrier()

        cv = cbuf[pl.ds(0, 16)]
        wev = cbuf[pl.ds(16, 16)]
        evbase = (g * NTILES + six) * (EPT * 16)
        exbase = ((cix * NG2 + r) * NTILES + six) * (EPT * 16)

        def p1(jg, c):
            pltpu.sync_copy(srcg.at[g, six, pl.ds(jg * GRP, GRP)], srcv8)
            pltpu.sync_copy(dstg.at[g, six, pl.ds(jg * GRP, GRP)], dgv8)
            pltpu.sync_copy(dstl.at[g, six, pl.ds(jg * GRP, GRP)], dlv8)

            def p1j(jj, cj):
                j = jg * GRP + jj
                pltpu.async_copy(sdt.at[srcv8.at[jj, 0]], srows, sem).wait()
                off = pl.multiple_of(evbase + j * (CH * 16), 8)
                pltpu.sync_copy(evb.at[pl.ds(off, CH * 16)], evv1)

                def pe_a(e2, c2):
                    ex16b[pl.ds(e2 * 16, 16)] = (
                        srows[e2, pl.ds(0, 16)]
                        + evv1[pl.ds(e2 * 16, 16)] * wev)
                    return c2
                lax.fori_loop(0, CH, pe_a, 0)
                pltpu.async_copy(sdt.at[dgv8.at[jj, 0]], srows, sem).wait()

                def pe(e2, c2):
                    a = ex16b[pl.ds(e2 * 16, 16)] + srows[e2, pl.ds(16, 16)]
                    a = jnp.where(a >= 0, a, 0.01 * a)
                    e_val = jnp.exp(a - cv)
                    msg[e2, pl.ds(0, 16)] = e_val
                    ex16b[pl.ds(e2 * 16, 16)] = e_val
                    return c2
                lax.fori_loop(0, CH, pe, 0)
                off2 = pl.multiple_of(exbase + j * (CH * 16), 8)
                pltpu.sync_copy(ex16b, exb.at[pl.ds(off2, CH * 16)])
                pltpu.sync_copy(msg, tab_sh.at[dlv8.at[jj, 0]], add=True)
                return cj
            lax.fori_loop(0, GRP, p1j, 0)
            return c
        lax.fori_loop(0, NCH // GRP, p1, 0)
        plsc.subcore_barrier()

        def dcp(k, c):
            base = pl.multiple_of(six * ROWS_PT + k * XCH, 8)
            pltpu.sync_copy(tab_sh.at[pl.ds(base, XCH)], xbuf)
            pltpu.sync_copy(xbuf, denb_c.at[pl.ds(base, XCH)])
            return c
        lax.fori_loop(0, ROWS_PT // XCH, dcp, 0)
        plsc.subcore_barrier()
        _zero_table()
        plsc.subcore_barrier()

        def p2(jg, c):
            pltpu.sync_copy(dstl.at[g, six, pl.ds(jg * GRP, GRP)], dlv8)

            def p2j(jj, cj):
                j = jg * GRP + jj
                pltpu.async_copy(denb_c.at[dlv8.at[jj, 0]], denrows, sem).wait()
                off = pl.multiple_of(exbase + j * (CH * 16), 8)
                pltpu.sync_copy(exb.at[pl.ds(off, CH * 16)], ex16b)
                pltpu.sync_copy(srch.at[g, six, pl.ds(j * (CH // CHH), CH // CHH)],
                                srchv)

                nsub = CH // CHH
                bufs = (hrows, hrows2)
                sems = (sem, sem2)
                desc = pltpu.async_copy(ht.at[srchv.at[0, 0]], bufs[0], sems[0])
                for kk in range(nsub):
                    cur = bufs[kk % 2]
                    if kk + 1 < nsub:
                        nxt_desc = pltpu.async_copy(
                            ht.at[srchv.at[kk + 1, 0]],
                            bufs[(kk + 1) % 2], sems[(kk + 1) % 2])
                    desc.wait()

                    def pm(e3, c2, kk=kk, cur=cur):
                        e2 = kk * CHH + e3
                        av = (ex16b[pl.ds(e2 * 16, 16)]
                              / denrows[e2, pl.ds(0, 16)] * 0.125)
                        accs = [jnp.zeros((16,), _f32) for _ in range(8)]
                        for h in range(H):
                            ah = av[h]
                            for k in range(8):
                                accs[k] = accs[k] + ah * cur[e3, pl.ds(h * F + k * 16, 16)]
                        for k in range(8):
                            msg[e2, pl.ds(k * 16, 16)] = accs[k]
                        return c2
                    lax.fori_loop(0, CHH, pm, 0)
                    if kk + 1 < nsub:
                        desc = nxt_desc
                pltpu.sync_copy(msg, tab_sh.at[dlv8.at[jj, 0]], add=True)
                return cj
            lax.fori_loop(0, GRP, p2j, 0)
            return c
        lax.fori_loop(0, NCH // GRP, p2, 0)
        plsc.subcore_barrier()

        def cpo(k, c):
            base = pl.multiple_of(six * ROWS_PT + k * XCH, 8)

            @pl.when(base < N)
            def _copy_out():
                pltpu.sync_copy(tab_sh.at[pl.ds(base, XCH)], xbuf)

                def lk(i, c2):
                    for kk in range(8):
                        v = xbuf[i, pl.ds(kk * 16, 16)]
                        xbuf[i, pl.ds(kk * 16, 16)] = jnp.where(v >= 0, v, 0.01 * v)
                    return c2
                lax.fori_loop(0, XCH, lk, 0)
                pltpu.sync_copy(xbuf, xn.at[pl.ds(pl.multiple_of(g * N + base, 8),
                                                  XCH)])
            return c
        lax.fori_loop(0, ROWS_PT // XCH, cpo, 0)
        plsc.subcore_barrier()
        return carry

    lax.fori_loop(0, NG2, round_body, 0)


def _edge_call(ht, sdt, srcg, dstg, dstl, srch, evb, cb):
    mesh = plsc.VectorSubcoreMesh(core_axis_name="c", subcore_axis_name="s",
                                  num_cores=NCORES, num_subcores=NTILES)
    fn = pl.kernel(
        _edge_body,
        out_type=[jax.ShapeDtypeStruct((G * N, F), _f32),
                  jax.ShapeDtypeStruct((NCORES * NG2 * NTILES * EPT * 16,), _f32),
                  jax.ShapeDtypeStruct((NCORES, NP, F), _f32)],
        mesh=mesh,
        scratch_types=[
            pltpu.VMEM_SHARED((NP, F), _f32),
            pltpu.VMEM((GRP, 1, CH), jnp.int32),
            pltpu.VMEM((GRP, 1, CH), jnp.int32),
            pltpu.VMEM((GRP, 1, CH), jnp.int32),
            pltpu.VMEM((CH // CHH, 1, CHH), jnp.int32),
            pltpu.VMEM((CH * 16,), _f32),
            pltpu.VMEM((CH * 16,), _f32),
            pltpu.VMEM((CH, F), _f32),
            pltpu.VMEM((CH, F), _f32),
            pltpu.VMEM((CHH, 1024), _f32),
            pltpu.VMEM((CHH, 1024), _f32),
            pltpu.VMEM((CH, F), _f32),
            pltpu.VMEM((XCH, F), _f32),
            pltpu.VMEM((32,), _f32),
            pltpu.SemaphoreType.DMA,
            pltpu.SemaphoreType.DMA,
        ],
    )
    return fn(ht, sdt, srcg, dstg, dstl, srch, evb, cb)


def _rsum_body(x_ref, o_ref):
    o_ref[...] = jnp.sum(x_ref[...], axis=0, keepdims=True)[None]


def _rsum(x):
    out = pl.pallas_call(
        _rsum_body,
        grid=(G,),
        in_specs=[pl.BlockSpec((N, F), lambda i: (i, 0))],
        out_specs=pl.BlockSpec((1, 1, F), lambda i: (i, 0, 0)),
        out_shape=jax.ShapeDtypeStruct((G, 1, F), _f32),
    )(x)
    return out.reshape(G, F)


def _head_body(sums_ref, wg1t_ref, bg1_ref, ratio_ref, wrt_ref, br_ref,
               temp_ref, wtt_ref, bt_ref, w1g_ref, w1r_ref, w1t_ref, b1_ref,
               w2t_ref, b2_ref, out_ref):
    hgs = _dot_hi(sums_ref[...], wg1t_ref[...]) + bg1_ref[...]
    hp = _dot_hi(hgs[0:1, :], w1g_ref[0])
    for g in range(1, G):
        hp = hp + _dot_hi(hgs[g:g + 1, :], w1g_ref[g])
    hr = _leaky_v(_dot_hi(ratio_ref[...], wrt_ref[...]) + br_ref[...])
    ht = _leaky_v(_dot_hi(temp_ref[...], wtt_ref[...]) + bt_ref[...])
    hp = hp + _dot_hi(hr, w1r_ref[...]) + _dot_hi(ht, w1t_ref[...])
    hidden = _leaky_v(hp + b1_ref[...])
    out_ref[...] = _dot_hi(hidden, w2t_ref[...]) + b2_ref[...]


def _head(sums, Wg1, bg1, ratio, Wr, br, temp, Wt, bt, W1, b1, W2, b2):
    w1g = W1[:, :G * 64].T.reshape(G, 64, 512)
    w1r = W1[:, G * 64:G * 64 + 6].T
    w1t = W1[:, G * 64 + 6:].T
    return pl.pallas_call(
        _head_body,
        out_shape=jax.ShapeDtypeStruct((1, 1), _f32),
    )(sums, Wg1.T, bg1.reshape(1, -1), ratio, Wr.T, br.reshape(1, -1),
      temp, Wt.T, bt.reshape(1, -1), w1g, w1r, w1t, b1.reshape(1, -1),
      W2.T, b2.reshape(1, 1))


def kernel(x1, edge_index1, e1, x2, edge_index2, e2, x3, edge_index3, e3,
           x4, edge_index4, e4, x5, edge_index5, e5, x6, edge_index6, e6,
           ratio, temp,
           Wn0, We0, as0, ad0, Wn1, We1, as1, ad1, Wn2, We2, as2, ad2,
           Wn3, We3, as3, ad3, Wn4, We4, as4, ad4, Wn5, We5, as5, ad5,
           Wg1, bg1, Wr, br, Wt, bt, W1, b1, W2, b2):
    kw = dict(locals())
    xs = [kw[f"x{g}"] for g in range(1, 7)]
    eis = [kw[f"edge_index{g}"] for g in range(1, 7)]
    evs = [kw[f"e{g}"] for g in range(1, 7)]

    pad_pt = EPT - E // NTILES

    def _ept_pad(a, val):
        a = a.reshape(G, NTILES, E // NTILES)
        return jnp.pad(a, ((0, 0), (0, 0), (0, pad_pt)), constant_values=val)

    src_p = _ept_pad(jnp.stack([eis[g][0] + g * N for g in range(G)]), 0)
    srcg = src_p.reshape(G, NTILES, NCH, 1, CH)
    srch = src_p.reshape(G, NTILES, EPT // CHH, 1, CHH)
    dstg = _ept_pad(jnp.stack([eis[g][1] + g * N for g in range(G)]), 0)
    dstg = dstg.reshape(G, NTILES, NCH, 1, CH)
    dstl = _ept_pad(jnp.stack([eis[g][1] for g in range(G)]), NFAKE)
    dstl = dstl.reshape(G, NTILES, NCH, 1, CH)
    ev_flat = jnp.stack(evs)
    ev_p = _ept_pad(ev_flat, 0.0)
    evb = jnp.broadcast_to(ev_p[..., None],
                           (G, NTILES, EPT, 16)).reshape(-1)

    x = jnp.concatenate(xs, axis=0)
    x = jnp.pad(x, ((0, 0), (0, 15)))

    for l in range(DEPTH):
        Wn, We = kw[f"Wn{l}"], kw[f"We{l}"]
        a_s, a_d = kw[f"as{l}"], kw[f"ad{l}"]
        dp = x.shape[1]
        d0 = Wn.shape[1]
        wn3 = Wn.reshape(H, F, d0)
        ws = jnp.einsum("hf,hfd->hd", a_s, wn3)
        wd = jnp.einsum("hf,hfd->hd", a_d, wn3)
        pad = ((0, dp - d0), (0, 0))
        wh = jnp.pad(Wn.T, pad)
        ws2 = jnp.concatenate([ws, ws], 0).T
        wd2 = jnp.concatenate([wd, wd], 0).T
        wsd = jnp.pad(jnp.concatenate([ws2, wd2], 1),
                      ((0, dp - d0), (0, F - 32)))
        we16 = jnp.tile(We[:, 0], 2)
        weabs = jnp.abs(we16).reshape(1, 16)

        ht, sd = _project(x, wh, wsd)
        cb8 = _bounds(sd, ev_flat, weabs)
        cb = jnp.concatenate(
            [cb8, jnp.broadcast_to(we16, (G, 16))], axis=1).reshape(G, 1, 32)
        x, _, _ = _edge_call(ht, sd, srcg, dstg, dstl, srch, evb, cb)

    sums = _rsum(x)
    return _head(sums, Wg1, bg1, ratio, Wr, br, temp, Wt, bt, W1, b1, W2, b2)


# speedup vs baseline: 15.3664x; 1.1179x over previous
import functools

import jax
import jax.numpy as jnp
from jax import lax
from jax.experimental import pallas as pl
from jax.experimental.pallas import tpu as pltpu
from jax.experimental.pallas import tpu_sc as plsc

N = 10000
NP = 10240
E = 160000
H = 8
F = 128
DEPTH = 6
G = 6
NCORES = 2
NTILES = 16
NG2 = G // NCORES
EPT = 10240
NFAKE = N + 64
CH = 64
NCH = EPT // CH
CHH = 8
GRP = 8
ROWS_PT = NP // NTILES
XCH = 8

_f32 = jnp.float32


def _leaky_v(x):
    return jnp.where(x >= 0, x, 0.01 * x)


def _dot_hi(a, b):
    return jax.lax.dot_general(a, b, (((1,), (0,)), ((), ())),
                               precision=jax.lax.Precision.HIGHEST,
                               preferred_element_type=_f32)


def _mm_body(x_ref, wh_ref, wsd_ref, h_ref, sd_ref):
    x = x_ref[...]
    h_ref[...] = _dot_hi(x, wh_ref[...])
    sd_ref[...] = _dot_hi(x, wsd_ref[...])


def _project(x, wh, wsd):
    m, dp = x.shape
    bm = 1000
    return pl.pallas_call(
        _mm_body,
        grid=(m // bm,),
        in_specs=[pl.BlockSpec((bm, dp), lambda i: (i, 0)),
                  pl.BlockSpec((dp, 1024), lambda i: (0, 0)),
                  pl.BlockSpec((dp, F), lambda i: (0, 0))],
        out_specs=[pl.BlockSpec((bm, 1024), lambda i: (i, 0)),
                   pl.BlockSpec((bm, F), lambda i: (i, 0))],
        out_shape=[jax.ShapeDtypeStruct((m, 1024), _f32),
                   jax.ShapeDtypeStruct((m, F), _f32)],
    )(x, wh, wsd)


def _bounds_body(sd_ref, e_ref, wa_ref, c_ref):
    m_all = jnp.max(sd_ref[...], axis=0, keepdims=True)
    ms = m_all[:, 0:16]
    md = m_all[:, 16:32]
    me = jnp.max(jnp.abs(e_ref[...]))
    c0 = ms + md + me * wa_ref[...]
    c_ref[...] = _leaky_v(c0)[None]


def _bounds(sd, ev, weabs):
    out = pl.pallas_call(
        _bounds_body,
        grid=(G,),
        in_specs=[pl.BlockSpec((N, F), lambda i: (i, 0)),
                  pl.BlockSpec((1, 1, E), lambda i: (i, 0, 0)),
                  pl.BlockSpec((1, 16), lambda i: (0, 0))],
        out_specs=pl.BlockSpec((1, 1, 16), lambda i: (i, 0, 0)),
        out_shape=jax.ShapeDtypeStruct((G, 1, 16), _f32),
    )(sd, ev.reshape(G, 1, E), weabs)
    return out.reshape(G, 16)


def _edge_body(ht, sdt, srcg, dstg, dstl, srch, evb, cb,
               xn, exb, denb,
               tab_sh,
               srcv8, dgv8, dlv8, srchv,
               evv1, ex16b,
               srows, denrows,
               hrows, hrows2, msg, xbuf, cbuf, sem, sem2, semd, seme):
    cix = lax.axis_index("c")
    six = lax.axis_index("s")
    denb_c = denb.at[cix]

    def _zero_xbuf():
        def zx(i, c):
            for k in range(8):
                xbuf[i, pl.ds(k * 16, 16)] = jnp.zeros((16,), _f32)
            return c
        lax.fori_loop(0, XCH, zx, 0)

    def _zero_table():
        _zero_xbuf()

        def zt(k, c):
            base = pl.multiple_of(six * ROWS_PT + k * XCH, 8)
            pltpu.sync_copy(xbuf, tab_sh.at[pl.ds(base, XCH)])
            return c
        lax.fori_loop(0, ROWS_PT // XCH, zt, 0)

    def round_body(r, carry):
        g = cix * NG2 + r
        pltpu.sync_copy(cb.at[g, 0], cbuf)
        _zero_table()

        def zm(i, c):
            for k in range(8):
                msg[i, pl.ds(k * 16, 16)] = jnp.zeros((16,), _f32)
            return c
        lax.fori_loop(0, CH, zm, 0)
        plsc.subcore_barrier()

        cv = cbuf[pl.ds(0, 16)]
        wev = cbuf[pl.ds(16, 16)]
        evbase = (g * NTILES + six) * (EPT * 16)
        exbase = ((cix * NG2 + r) * NTILES + six) * (EPT * 16)

        def p1(jg, c):
            pltpu.sync_copy(srcg.at[g, six, pl.ds(jg * GRP, GRP)], srcv8)
            pltpu.sync_copy(dstg.at[g, six, pl.ds(jg * GRP, GRP)], dgv8)
            pltpu.sync_copy(dstl.at[g, six, pl.ds(jg * GRP, GRP)], dlv8)

            def p1j(jj, cj):
                j = jg * GRP + jj
                dsc = pltpu.async_copy(sdt.at[srcv8.at[jj, 0]], srows, sem)
                off = pl.multiple_of(evbase + j * (CH * 16), 8)
                dev = pltpu.async_copy(evb.at[pl.ds(off, CH * 16)], evv1, seme)
                dsc.wait()
                dev.wait()

                def pe_a(e2, c2):
                    ex16b[pl.ds(e2 * 16, 16)] = (
                        srows[e2, pl.ds(0, 16)]
                        + evv1[pl.ds(e2 * 16, 16)] * wev)
                    return c2
                lax.fori_loop(0, CH, pe_a, 0)
                pltpu.async_copy(sdt.at[dgv8.at[jj, 0]], srows, sem).wait()

                def pe(e2, c2):
                    a = ex16b[pl.ds(e2 * 16, 16)] + srows[e2, pl.ds(16, 16)]
                    a = jnp.where(a >= 0, a, 0.01 * a)
                    e_val = jnp.exp(a - cv)
                    msg[e2, pl.ds(0, 16)] = e_val
                    ex16b[pl.ds(e2 * 16, 16)] = e_val
                    return c2
                lax.fori_loop(0, CH, pe, 0)
                off2 = pl.multiple_of(exbase + j * (CH * 16), 8)
                pltpu.sync_copy(ex16b, exb.at[pl.ds(off2, CH * 16)])
                pltpu.sync_copy(msg, tab_sh.at[dlv8.at[jj, 0]], add=True)
                return cj
            lax.fori_loop(0, GRP, p1j, 0)
            return c
        lax.fori_loop(0, NCH // GRP, p1, 0)
        plsc.subcore_barrier()

        def dcp(k, c):
            base = pl.multiple_of(six * ROWS_PT + k * XCH, 8)
            pltpu.sync_copy(tab_sh.at[pl.ds(base, XCH)], xbuf)
            pltpu.sync_copy(xbuf, denb_c.at[pl.ds(base, XCH)])
            return c
        lax.fori_loop(0, ROWS_PT // XCH, dcp, 0)
        plsc.subcore_barrier()
        _zero_table()
        plsc.subcore_barrier()

        def p2(jg, c):
            pltpu.sync_copy(dstl.at[g, six, pl.ds(jg * GRP, GRP)], dlv8)

            def p2j(jj, cj):
                j = jg * GRP + jj
                dd = pltpu.async_copy(denb_c.at[dlv8.at[jj, 0]], denrows, semd)
                off = pl.multiple_of(exbase + j * (CH * 16), 8)
                de = pltpu.async_copy(exb.at[pl.ds(off, CH * 16)], ex16b, seme)
                pltpu.sync_copy(srch.at[g, six, pl.ds(j * (CH // CHH), CH // CHH)],
                                srchv)

                nsub = CH // CHH
                bufs = (hrows, hrows2)
                sems = (sem, sem2)
                desc = pltpu.async_copy(ht.at[srchv.at[0, 0]], bufs[0], sems[0])
                dd.wait()
                de.wait()
                for kk in range(nsub):
                    cur = bufs[kk % 2]
                    if kk + 1 < nsub:
                        nxt_desc = pltpu.async_copy(
                            ht.at[srchv.at[kk + 1, 0]],
                            bufs[(kk + 1) % 2], sems[(kk + 1) % 2])
                    desc.wait()

                    def pm(e3, c2, kk=kk, cur=cur):
                        e2 = kk * CHH + e3
                        av = (ex16b[pl.ds(e2 * 16, 16)]
                              / denrows[e2, pl.ds(0, 16)] * 0.125)
                        accs = [jnp.zeros((16,), _f32) for _ in range(8)]
                        for h in range(H):
                            ah = av[h]
                            for k in range(8):
                                accs[k] = accs[k] + ah * cur[e3, pl.ds(h * F + k * 16, 16)]
                        for k in range(8):
                            msg[e2, pl.ds(k * 16, 16)] = accs[k]
                        return c2
                    lax.fori_loop(0, CHH, pm, 0)
                    if kk + 1 < nsub:
                        desc = nxt_desc
                pltpu.sync_copy(msg, tab_sh.at[dlv8.at[jj, 0]], add=True)
                return cj
            lax.fori_loop(0, GRP, p2j, 0)
            return c
        lax.fori_loop(0, NCH // GRP, p2, 0)
        plsc.subcore_barrier()

        def cpo(k, c):
            base = pl.multiple_of(six * ROWS_PT + k * XCH, 8)

            @pl.when(base < N)
            def _copy_out():
                pltpu.sync_copy(tab_sh.at[pl.ds(base, XCH)], xbuf)

                def lk(i, c2):
                    for kk in range(8):
                        v = xbuf[i, pl.ds(kk * 16, 16)]
                        xbuf[i, pl.ds(kk * 16, 16)] = jnp.where(v >= 0, v, 0.01 * v)
                    return c2
                lax.fori_loop(0, XCH, lk, 0)
                pltpu.sync_copy(xbuf, xn.at[pl.ds(pl.multiple_of(g * N + base, 8),
                                                  XCH)])
            return c
        lax.fori_loop(0, ROWS_PT // XCH, cpo, 0)
        plsc.subcore_barrier()
        return carry

    lax.fori_loop(0, NG2, round_body, 0)


def _edge_call(ht, sdt, srcg, dstg, dstl, srch, evb, cb):
    mesh = plsc.VectorSubcoreMesh(core_axis_name="c", subcore_axis_name="s",
                                  num_cores=NCORES, num_subcores=NTILES)
    fn = pl.kernel(
        _edge_body,
        out_type=[jax.ShapeDtypeStruct((G * N, F), _f32),
                  jax.ShapeDtypeStruct((NCORES * NG2 * NTILES * EPT * 16,), _f32),
                  jax.ShapeDtypeStruct((NCORES, NP, F), _f32)],
        mesh=mesh,
        scratch_types=[
            pltpu.VMEM_SHARED((NP, F), _f32),
            pltpu.VMEM((GRP, 1, CH), jnp.int32),
            pltpu.VMEM((GRP, 1, CH), jnp.int32),
            pltpu.VMEM((GRP, 1, CH), jnp.int32),
            pltpu.VMEM((CH // CHH, 1, CHH), jnp.int32),
            pltpu.VMEM((CH * 16,), _f32),
            pltpu.VMEM((CH * 16,), _f32),
            pltpu.VMEM((CH, F), _f32),
            pltpu.VMEM((CH, F), _f32),
            pltpu.VMEM((CHH, 1024), _f32),
            pltpu.VMEM((CHH, 1024), _f32),
            pltpu.VMEM((CH, F), _f32),
            pltpu.VMEM((XCH, F), _f32),
            pltpu.VMEM((32,), _f32),
            pltpu.SemaphoreType.DMA,
            pltpu.SemaphoreType.DMA,
            pltpu.SemaphoreType.DMA,
            pltpu.SemaphoreType.DMA,
        ],
    )
    return fn(ht, sdt, srcg, dstg, dstl, srch, evb, cb)


def _rsum_body(x_ref, o_ref):
    o_ref[...] = jnp.sum(x_ref[...], axis=0, keepdims=True)[None]


def _rsum(x):
    out = pl.pallas_call(
        _rsum_body,
        grid=(G,),
        in_specs=[pl.BlockSpec((N, F), lambda i: (i, 0))],
        out_specs=pl.BlockSpec((1, 1, F), lambda i: (i, 0, 0)),
        out_shape=jax.ShapeDtypeStruct((G, 1, F), _f32),
    )(x)
    return out.reshape(G, F)


def _head_body(sums_ref, wg1t_ref, bg1_ref, ratio_ref, wrt_ref, br_ref,
               temp_ref, wtt_ref, bt_ref, w1g_ref, w1r_ref, w1t_ref, b1_ref,
               w2t_ref, b2_ref, out_ref):
    hgs = _dot_hi(sums_ref[...], wg1t_ref[...]) + bg1_ref[...]
    hp = _dot_hi(hgs[0:1, :], w1g_ref[0])
    for g in range(1, G):
        hp = hp + _dot_hi(hgs[g:g + 1, :], w1g_ref[g])
    hr = _leaky_v(_dot_hi(ratio_ref[...], wrt_ref[...]) + br_ref[...])
    ht = _leaky_v(_dot_hi(temp_ref[...], wtt_ref[...]) + bt_ref[...])
    hp = hp + _dot_hi(hr, w1r_ref[...]) + _dot_hi(ht, w1t_ref[...])
    hidden = _leaky_v(hp + b1_ref[...])
    out_ref[...] = _dot_hi(hidden, w2t_ref[...]) + b2_ref[...]


def _head(sums, Wg1, bg1, ratio, Wr, br, temp, Wt, bt, W1, b1, W2, b2):
    w1g = W1[:, :G * 64].T.reshape(G, 64, 512)
    w1r = W1[:, G * 64:G * 64 + 6].T
    w1t = W1[:, G * 64 + 6:].T
    return pl.pallas_call(
        _head_body,
        out_shape=jax.ShapeDtypeStruct((1, 1), _f32),
    )(sums, Wg1.T, bg1.reshape(1, -1), ratio, Wr.T, br.reshape(1, -1),
      temp, Wt.T, bt.reshape(1, -1), w1g, w1r, w1t, b1.reshape(1, -1),
      W2.T, b2.reshape(1, 1))


def kernel(x1, edge_index1, e1, x2, edge_index2, e2, x3, edge_index3, e3,
           x4, edge_index4, e4, x5, edge_index5, e5, x6, edge_index6, e6,
           ratio, temp,
           Wn0, We0, as0, ad0, Wn1, We1, as1, ad1, Wn2, We2, as2, ad2,
           Wn3, We3, as3, ad3, Wn4, We4, as4, ad4, Wn5, We5, as5, ad5,
           Wg1, bg1, Wr, br, Wt, bt, W1, b1, W2, b2):
    kw = dict(locals())
    xs = [kw[f"x{g}"] for g in range(1, 7)]
    eis = [kw[f"edge_index{g}"] for g in range(1, 7)]
    evs = [kw[f"e{g}"] for g in range(1, 7)]

    pad_pt = EPT - E // NTILES

    def _ept_pad(a, val):
        a = a.reshape(G, NTILES, E // NTILES)
        return jnp.pad(a, ((0, 0), (0, 0), (0, pad_pt)), constant_values=val)

    src_p = _ept_pad(jnp.stack([eis[g][0] + g * N for g in range(G)]), 0)
    srcg = src_p.reshape(G, NTILES, NCH, 1, CH)
    srch = src_p.reshape(G, NTILES, EPT // CHH, 1, CHH)
    dstg = _ept_pad(jnp.stack([eis[g][1] + g * N for g in range(G)]), 0)
    dstg = dstg.reshape(G, NTILES, NCH, 1, CH)
    dstl = _ept_pad(jnp.stack([eis[g][1] for g in range(G)]), NFAKE)
    dstl = dstl.reshape(G, NTILES, NCH, 1, CH)
    ev_flat = jnp.stack(evs)
    ev_p = _ept_pad(ev_flat, 0.0)
    evb = jnp.broadcast_to(ev_p[..., None],
                           (G, NTILES, EPT, 16)).reshape(-1)

    x = jnp.concatenate(xs, axis=0)
    x = jnp.pad(x, ((0, 0), (0, 15)))

    for l in range(DEPTH):
        Wn, We = kw[f"Wn{l}"], kw[f"We{l}"]
        a_s, a_d = kw[f"as{l}"], kw[f"ad{l}"]
        dp = x.shape[1]
        d0 = Wn.shape[1]
        wn3 = Wn.reshape(H, F, d0)
        ws = jnp.einsum("hf,hfd->hd", a_s, wn3)
        wd = jnp.einsum("hf,hfd->hd", a_d, wn3)
        pad = ((0, dp - d0), (0, 0))
        wh = jnp.pad(Wn.T, pad)
        ws2 = jnp.concatenate([ws, ws], 0).T
        wd2 = jnp.concatenate([wd, wd], 0).T
        wsd = jnp.pad(jnp.concatenate([ws2, wd2], 1),
                      ((0, dp - d0), (0, F - 32)))
        we16 = jnp.tile(We[:, 0], 2)
        weabs = jnp.abs(we16).reshape(1, 16)

        ht, sd = _project(x, wh, wsd)
        cb8 = _bounds(sd, ev_flat, weabs)
        cb = jnp.concatenate(
            [cb8, jnp.broadcast_to(we16, (G, 16))], axis=1).reshape(G, 1, 32)
        x, _, _ = _edge_call(ht, sd, srcg, dstg, dstl, srch, evb, cb)

    sums = _rsum(x)
    return _head(sums, Wg1, bg1, ratio, Wr, br, temp, Wt, bt, W1, b1, W2, b2)


# speedup vs baseline: 15.3670x; 1.0000x over previous
import functools

import jax
import jax.numpy as jnp
from jax import lax
from jax.experimental import pallas as pl
from jax.experimental.pallas import tpu as pltpu
from jax.experimental.pallas import tpu_sc as plsc

N = 10000
NP = 10240
E = 160000
H = 8
F = 128
DEPTH = 6
G = 6
NCORES = 2
NTILES = 16
NG2 = G // NCORES
EPT = 10240
NFAKE = N + 64
CH = 64
NCH = EPT // CH
CHH = 8
GRP = 8
ROWS_PT = NP // NTILES
XCH = 8

_f32 = jnp.float32


def _leaky_v(x):
    return jnp.where(x >= 0, x, 0.01 * x)


def _dot_hi(a, b):
    return jax.lax.dot_general(a, b, (((1,), (0,)), ((), ())),
                               precision=jax.lax.Precision.HIGHEST,
                               preferred_element_type=_f32)


def _mm_body(x_ref, wh_ref, wsd_ref, h_ref, sd_ref):
    x = x_ref[...]
    h_ref[...] = _dot_hi(x, wh_ref[...])
    sd_ref[...] = _dot_hi(x, wsd_ref[...])


def _project(x, wh, wsd):
    m, dp = x.shape
    bm = 1000
    return pl.pallas_call(
        _mm_body,
        grid=(m // bm,),
        in_specs=[pl.BlockSpec((bm, dp), lambda i: (i, 0)),
                  pl.BlockSpec((dp, 1024), lambda i: (0, 0)),
                  pl.BlockSpec((dp, F), lambda i: (0, 0))],
        out_specs=[pl.BlockSpec((bm, 1024), lambda i: (i, 0)),
                   pl.BlockSpec((bm, F), lambda i: (i, 0))],
        out_shape=[jax.ShapeDtypeStruct((m, 1024), _f32),
                   jax.ShapeDtypeStruct((m, F), _f32)],
    )(x, wh, wsd)


def _bounds_body(sd_ref, e_ref, wa_ref, c_ref):
    m_all = jnp.max(sd_ref[...], axis=0, keepdims=True)
    ms = m_all[:, 0:16]
    md = m_all[:, 16:32]
    me = jnp.max(jnp.abs(e_ref[...]))
    c0 = ms + md + me * wa_ref[...]
    c_ref[...] = _leaky_v(c0)[None]


def _bounds(sd, ev, weabs):
    out = pl.pallas_call(
        _bounds_body,
        grid=(G,),
        in_specs=[pl.BlockSpec((N, F), lambda i: (i, 0)),
                  pl.BlockSpec((1, 1, E), lambda i: (i, 0, 0)),
                  pl.BlockSpec((1, 16), lambda i: (0, 0))],
        out_specs=pl.BlockSpec((1, 1, 16), lambda i: (i, 0, 0)),
        out_shape=jax.ShapeDtypeStruct((G, 1, 16), _f32),
    )(sd, ev.reshape(G, 1, E), weabs)
    return out.reshape(G, 16)


def _edge_body(ht, sdt, srcg, dstg, dstl, srch, evb, cb,
               xn, exb, denb,
               tab_sh,
               srcv8, dgv8, dlv8, srchv,
               evv1, ex16b,
               srows, denrows,
               hrows, hrows2, msg, xbuf, cbuf, sem, sem2, semd, seme):
    cix = lax.axis_index("c")
    six = lax.axis_index("s")
    denb_c = denb.at[cix]

    def _zero_xbuf():
        def zx(i, c):
            for k in range(8):
                xbuf[i, pl.ds(k * 16, 16)] = jnp.zeros((16,), _f32)
            return c
        lax.fori_loop(0, XCH, zx, 0)

    def _zero_table():
        _zero_xbuf()

        def zt(k, c):
            base = pl.multiple_of(six * ROWS_PT + k * XCH, 8)
            pltpu.sync_copy(xbuf, tab_sh.at[pl.ds(base, XCH)])
            return c
        lax.fori_loop(0, ROWS_PT // XCH, zt, 0)

    def round_body(r, carry):
        g = cix * NG2 + r
        pltpu.sync_copy(cb.at[g, 0], cbuf)
        _zero_table()

        def zm(i, c):
            for k in range(8):
                msg[i, pl.ds(k * 16, 16)] = jnp.zeros((16,), _f32)
            return c
        lax.fori_loop(0, CH, zm, 0)
        plsc.subcore_barrier()

        cv = cbuf[pl.ds(0, 16)]
        wev = cbuf[pl.ds(16, 16)]
        evbase = (g * NTILES + six) * (EPT * 16)
        exbase = ((cix * NG2 + r) * NTILES + six) * (EPT * 16)

        def p1(jg, c):
            pltpu.sync_copy(srcg.at[g, six, pl.ds(jg * GRP, GRP)], srcv8)
            pltpu.sync_copy(dstg.at[g, six, pl.ds(jg * GRP, GRP)], dgv8)
            pltpu.sync_copy(dstl.at[g, six, pl.ds(jg * GRP, GRP)], dlv8)

            def p1j(jj, cj):
                j = jg * GRP + jj
                dsc = pltpu.async_copy(sdt.at[srcv8.at[jj, 0]], srows, sem)
                off = pl.multiple_of(evbase + j * (CH * 16), 8)
                dev = pltpu.async_copy(evb.at[pl.ds(off, CH * 16)], evv1, seme)
                dsc.wait()
                dev.wait()

                def pe_a(e2, c2):
                    ex16b[pl.ds(e2 * 16, 16)] = (
                        srows[e2, pl.ds(0, 16)]
                        + evv1[pl.ds(e2 * 16, 16)] * wev)
                    return c2
                lax.fori_loop(0, CH, pe_a, 0)
                pltpu.async_copy(sdt.at[dgv8.at[jj, 0]], srows, sem).wait()

                def pe(e2, c2):
                    a = ex16b[pl.ds(e2 * 16, 16)] + srows[e2, pl.ds(16, 16)]
                    a = jnp.where(a >= 0, a, 0.01 * a)
                    e_val = jnp.exp(a - cv)
                    msg[e2, pl.ds(0, 16)] = e_val
                    ex16b[pl.ds(e2 * 16, 16)] = e_val
                    return c2
                lax.fori_loop(0, CH, pe, 0)
                off2 = pl.multiple_of(exbase + j * (CH * 16), 8)
                pltpu.sync_copy(ex16b, exb.at[pl.ds(off2, CH * 16)])
                pltpu.sync_copy(msg, tab_sh.at[dlv8.at[jj, 0]], add=True)
                return cj
            lax.fori_loop(0, GRP, p1j, 0)
            return c
        lax.fori_loop(0, NCH // GRP, p1, 0)
        plsc.subcore_barrier()

        def dcp(k, c):
            base = pl.multiple_of(six * ROWS_PT + k * XCH, 8)
            pltpu.sync_copy(tab_sh.at[pl.ds(base, XCH)], xbuf)
            pltpu.sync_copy(xbuf, denb_c.at[pl.ds(base, XCH)])
            return c
        lax.fori_loop(0, ROWS_PT // XCH, dcp, 0)
        plsc.subcore_barrier()
        _zero_table()
        plsc.subcore_barrier()

        def p2(jg, c):
            pltpu.sync_copy(dstl.at[g, six, pl.ds(jg * GRP, GRP)], dlv8)

            def p2j(jj, cj):
                j = jg * GRP + jj
                dd = pltpu.async_copy(denb_c.at[dlv8.at[jj, 0]], denrows, semd)
                off = pl.multiple_of(exbase + j * (CH * 16), 8)
                de = pltpu.async_copy(exb.at[pl.ds(off, CH * 16)], ex16b, seme)
                pltpu.sync_copy(srch.at[g, six, pl.ds(j * (CH // CHH), CH // CHH)],
                                srchv)

                nsub = CH // CHH
                bufs = (hrows, hrows2)
                sems = (sem, sem2)
                desc = pltpu.async_copy(ht.at[srchv.at[0, 0]], bufs[0], sems[0])
                dd.wait()
                de.wait()
                for kk in range(nsub):
                    cur = bufs[kk % 2]
                    if kk + 1 < nsub:
                        nxt_desc = pltpu.async_copy(
                            ht.at[srchv.at[kk + 1, 0]],
                            bufs[(kk + 1) % 2], sems[(kk + 1) % 2])
                    desc.wait()

                    def pm(e3, c2, kk=kk, cur=cur):
                        e2 = kk * CHH + e3
                        av = (ex16b[pl.ds(e2 * 16, 16)]
                              / denrows[e2, pl.ds(0, 16)] * 0.125)
                        accs = [jnp.zeros((16,), _f32) for _ in range(8)]
                        for h in range(H):
                            ah = av[h]
                            for k in range(8):
                                accs[k] = accs[k] + ah * cur[e3, pl.ds(h * F + k * 16, 16)]
                        for k in range(8):
                            msg[e2, pl.ds(k * 16, 16)] = accs[k]
                        return c2
                    lax.fori_loop(0, CHH, pm, 0)
                    if kk + 1 < nsub:
                        desc = nxt_desc
                pltpu.sync_copy(msg, tab_sh.at[dlv8.at[jj, 0]], add=True)
                return cj
            lax.fori_loop(0, GRP, p2j, 0)
            return c
        lax.fori_loop(0, NCH // GRP, p2, 0)
        plsc.subcore_barrier()

        def cpo(k, c):
            base = pl.multiple_of(six * ROWS_PT + k * XCH, 8)

            @pl.when(base < N)
            def _copy_out():
                pltpu.sync_copy(tab_sh.at[pl.ds(base, XCH)], xbuf)

                def lk(i, c2):
                    for kk in range(8):
                        v = xbuf[i, pl.ds(kk * 16, 16)]
                        xbuf[i, pl.ds(kk * 16, 16)] = jnp.where(v >= 0, v, 0.01 * v)
                    return c2
                lax.fori_loop(0, XCH, lk, 0)
                pltpu.sync_copy(xbuf, xn.at[pl.ds(pl.multiple_of(g * N + base, 8),
                                                  XCH)])
            return c
        lax.fori_loop(0, ROWS_PT // XCH, cpo, 0)
        plsc.subcore_barrier()
        return carry

    lax.fori_loop(0, NG2, round_body, 0)


def _edge_call(ht, sdt, srcg, dstg, dstl, srch, evb, cb):
    mesh = plsc.VectorSubcoreMesh(core_axis_name="c", subcore_axis_name="s",
                                  num_cores=NCORES, num_subcores=NTILES)
    fn = pl.kernel(
        _edge_body,
        out_type=[jax.ShapeDtypeStruct((G * N, F), _f32),
                  jax.ShapeDtypeStruct((NCORES * NG2 * NTILES * EPT * 16,), _f32),
                  jax.ShapeDtypeStruct((NCORES, NP, F), _f32)],
        mesh=mesh,
        scratch_types=[
            pltpu.VMEM_SHARED((NP, F), _f32),
            pltpu.VMEM((GRP, 1, CH), jnp.int32),
            pltpu.VMEM((GRP, 1, CH), jnp.int32),
            pltpu.VMEM((GRP, 1, CH), jnp.int32),
            pltpu.VMEM((CH // CHH, 1, CHH), jnp.int32),
            pltpu.VMEM((CH * 16,), _f32),
            pltpu.VMEM((CH * 16,), _f32),
            pltpu.VMEM((CH, F), _f32),
            pltpu.VMEM((CH, F), _f32),
            pltpu.VMEM((CHH, 1024), _f32),
            pltpu.VMEM((CHH, 1024), _f32),
            pltpu.VMEM((CH, F), _f32),
            pltpu.VMEM((XCH, F), _f32),
            pltpu.VMEM((32,), _f32),
            pltpu.SemaphoreType.DMA,
            pltpu.SemaphoreType.DMA,
            pltpu.SemaphoreType.DMA,
            pltpu.SemaphoreType.DMA,
        ],
    )
    return fn(ht, sdt, srcg, dstg, dstl, srch, evb, cb)


def _rsum_body(x_ref, o_ref):
    o_ref[...] = jnp.sum(x_ref[...], axis=0, keepdims=True)[None]


def _rsum(x):
    out = pl.pallas_call(
        _rsum_body,
        grid=(G,),
        in_specs=[pl.BlockSpec((N, F), lambda i: (i, 0))],
        out_specs=pl.BlockSpec((1, 1, F), lambda i: (i, 0, 0)),
        out_shape=jax.ShapeDtypeStruct((G, 1, F), _f32),
    )(x)
    return out.reshape(G, F)


def _head_body(sums_ref, wg1t_ref, bg1_ref, ratio_ref, wrt_ref, br_ref,
               temp_ref, wtt_ref, bt_ref, w1g_ref, w1r_ref, w1t_ref, b1_ref,
               w2t_ref, b2_ref, out_ref):
    hgs = _dot_hi(sums_ref[...], wg1t_ref[...]) + bg1_ref[...]
    hp = _dot_hi(hgs[0:1, :], w1g_ref[0])
    for g in range(1, G):
        hp = hp + _dot_hi(hgs[g:g + 1, :], w1g_ref[g])
    hr = _leaky_v(_dot_hi(ratio_ref[...], wrt_ref[...]) + br_ref[...])
    ht = _leaky_v(_dot_hi(temp_ref[...], wtt_ref[...]) + bt_ref[...])
    hp = hp + _dot_hi(hr, w1r_ref[...]) + _dot_hi(ht, w1t_ref[...])
    hidden = _leaky_v(hp + b1_ref[...])
    out_ref[...] = _dot_hi(hidden, w2t_ref[...]) + b2_ref[...]


def _head(sums, Wg1, bg1, ratio, Wr, br, temp, Wt, bt, W1, b1, W2, b2):
    w1g = W1[:, :G * 64].T.reshape(G, 64, 512)
    w1r = W1[:, G * 64:G * 64 + 6].T
    w1t = W1[:, G * 64 + 6:].T
    return pl.pallas_call(
        _head_body,
        out_shape=jax.ShapeDtypeStruct((1, 1), _f32),
    )(sums, Wg1.T, bg1.reshape(1, -1), ratio, Wr.T, br.reshape(1, -1),
      temp, Wt.T, bt.reshape(1, -1), w1g, w1r, w1t, b1.reshape(1, -1),
      W2.T, b2.reshape(1, 1))


def kernel(x1, edge_index1, e1, x2, edge_index2, e2, x3, edge_index3, e3,
           x4, edge_index4, e4, x5, edge_index5, e5, x6, edge_index6, e6,
           ratio, temp,
           Wn0, We0, as0, ad0, Wn1, We1, as1, ad1, Wn2, We2, as2, ad2,
           Wn3, We3, as3, ad3, Wn4, We4, as4, ad4, Wn5, We5, as5, ad5,
           Wg1, bg1, Wr, br, Wt, bt, W1, b1, W2, b2):
    kw = dict(locals())
    xs = [kw[f"x{g}"] for g in range(1, 7)]
    eis = [kw[f"edge_index{g}"] for g in range(1, 7)]
    evs = [kw[f"e{g}"] for g in range(1, 7)]

    pad_pt = EPT - E // NTILES

    def _ept_pad(a, val):
        a = a.reshape(G, NTILES, E // NTILES)
        return jnp.pad(a, ((0, 0), (0, 0), (0, pad_pt)), constant_values=val)

    src_p = _ept_pad(jnp.stack([eis[g][0] + g * N for g in range(G)]), 0)
    srcg = src_p.reshape(G, NTILES, NCH, 1, CH)
    srch = src_p.reshape(G, NTILES, EPT // CHH, 1, CHH)
    dstg = _ept_pad(jnp.stack([eis[g][1] + g * N for g in range(G)]), 0)
    dstg = dstg.reshape(G, NTILES, NCH, 1, CH)
    dstl = _ept_pad(jnp.stack([eis[g][1] for g in range(G)]), NFAKE)
    dstl = dstl.reshape(G, NTILES, NCH, 1, CH)
    ev_flat = jnp.stack(evs)
    ev_p = _ept_pad(ev_flat, 0.0)
    evb = jnp.broadcast_to(ev_p[..., None],
                           (G, NTILES, EPT, 16)).reshape(-1)

    x = jnp.concatenate(xs, axis=0)
    x = jnp.pad(x, ((0, 0), (0, 15)))

    for l in range(DEPTH):
        Wn, We = kw[f"Wn{l}"], kw[f"We{l}"]
        a_s, a_d = kw[f"as{l}"], kw[f"ad{l}"]
        dp = x.shape[1]
        d0 = Wn.shape[1]
        wn3 = Wn.reshape(H, F, d0)
        ws = jnp.einsum("hf,hfd->hd", a_s, wn3,
                        precision=jax.lax.Precision.HIGHEST)
        wd = jnp.einsum("hf,hfd->hd", a_d, wn3,
                        precision=jax.lax.Precision.HIGHEST)
        pad = ((0, dp - d0), (0, 0))
        wh = jnp.pad(Wn.T, pad)
        ws2 = jnp.concatenate([ws, ws], 0).T
        wd2 = jnp.concatenate([wd, wd], 0).T
        wsd = jnp.pad(jnp.concatenate([ws2, wd2], 1),
                      ((0, dp - d0), (0, F - 32)))
        we16 = jnp.tile(We[:, 0], 2)
        weabs = jnp.abs(we16).reshape(1, 16)

        ht, sd = _project(x, wh, wsd)
        cb8 = _bounds(sd, ev_flat, weabs)
        cb = jnp.concatenate(
            [cb8, jnp.broadcast_to(we16, (G, 16))], axis=1).reshape(G, 1, 32)
        x, _, _ = _edge_call(ht, sd, srcg, dstg, dstl, srch, evb, cb)

    sums = _rsum(x)
    return _head(sums, Wg1, bg1, ratio, Wr, br, temp, Wt, bt, W1, b1, W2, b2)
